# Initial kernel scaffold; baseline (speedup 1.0000x reference)
import jax
import jax.numpy as jnp
from jax.experimental import pallas as pl


def kernel(x, edge_index, batch, W0, asrc0, adst0, b0, W1, asrc1, adst1, b1, W2, asrc2, adst2, b2, W3, asrc3, adst3, b3):
    raise NotImplementedError("write your pallas kernel here")


# speedup vs baseline: 32.3632x; 32.3632x over previous
import functools

import jax
import jax.numpy as jnp
from jax import lax
from jax.experimental import pallas as pl
from jax.experimental.pallas import tpu as pltpu
from jax.experimental.pallas import tpu_sc as plsc

N = 10000
G = 16
HID = 128
NC, NS, L = 2, 16, 16
NW = NC * NS
NP = 10240
E_TOT = 320000 + N
BB = 128
NB = 81
EC = NB * BB
EPAD = NW * EC
D = HID

_GDN = lax.GatherDimensionNumbers(offset_dims=(), collapsed_slice_dims=(0,),
                                  start_index_map=(0,))


def _splat(vec, lane):
    idx = jnp.full((L, 1), lane, jnp.int32)
    return lax.gather(vec, idx, _GDN, (1,),
                      mode=lax.GatherScatterMode.PROMISE_IN_BOUNDS)


def _mm_body(x_ref, w_ref, a_ref, h_ref, al_ref):
    h = jnp.dot(x_ref[...], w_ref[...], preferred_element_type=jnp.float32)
    h_ref[...] = h
    al_ref[...] = jnp.dot(h, a_ref[...], preferred_element_type=jnp.float32)


def _dense(x, W, A):
    n, din = x.shape
    dout = W.shape[1]
    bn = 2000
    return pl.pallas_call(
        _mm_body,
        grid=(n // bn,),
        in_specs=[
            pl.BlockSpec((bn, din), lambda i: (i, 0)),
            pl.BlockSpec((din, dout), lambda i: (0, 0)),
            pl.BlockSpec((dout, A.shape[1]), lambda i: (0, 0)),
        ],
        out_specs=[
            pl.BlockSpec((bn, dout), lambda i: (i, 0)),
            pl.BlockSpec((bn, A.shape[1]), lambda i: (i, 0)),
        ],
        out_shape=[
            jax.ShapeDtypeStruct((n, dout), jnp.float32),
            jax.ShapeDtypeStruct((n, A.shape[1]), jnp.float32),
        ],
    )(x, W, A)


_mesh = plsc.VectorSubcoreMesh(core_axis_name="c", subcore_axis_name="s")
_RPT = NP // NS
_WCH = 128
DH = 64


@functools.partial(
    pl.kernel,
    out_type=[
        jax.ShapeDtypeStruct((NC, NP, DH), jnp.float32),
        jax.ShapeDtypeStruct((NW, NP), jnp.float32),
    ],
    mesh=_mesh,
    scratch_types=[
        pltpu.VMEM((NB, BB), jnp.int32),
        pltpu.VMEM((NB, BB), jnp.int32),
        pltpu.VMEM((NP,), jnp.float32),
        pltpu.VMEM((NP,), jnp.float32),
        pltpu.VMEM((NP,), jnp.float32),
        pltpu.VMEM((NP,), jnp.float32),
        pltpu.VMEM((BB, DH), jnp.float32),
        pltpu.VMEM_SHARED((NP, DH), jnp.float32),
        pltpu.SemaphoreType.DMA,
    ],
    compiler_params=pltpu.CompilerParams(needs_layout_passes=False, use_tc_tiling_on_sc=False),
)
def _edge_kernel(src_hbm, dst_hbm, asrc_hbm, adst_hbm, c_hbm, h_hbm,
                 agg_hbm, dsum_hbm,
                 src_v, dst_v, asrc_v, adst_v, c_v, dloc_v, rows_v, acc_sh, sem):
    cid = lax.axis_index("c")
    sid = lax.axis_index("s")
    wid = sid * NC + cid
    pltpu.sync_copy(src_hbm.at[wid], src_v)
    pltpu.sync_copy(dst_hbm.at[wid], dst_v)
    pltpu.sync_copy(asrc_hbm, asrc_v)
    pltpu.sync_copy(adst_hbm, adst_v)
    pltpu.sync_copy(c_hbm, c_v)

    z16 = jnp.zeros((L,), jnp.float32)
    for i in range(_WCH):
        for f in range(DH // L):
            rows_v[i, pl.ds(f * L, L)] = z16
    for i in range(NP // L):
        dloc_v[pl.ds(i * L, L)] = z16
    base = sid * _RPT
    for k in range(0, _RPT, _WCH):
        sz = min(_WCH, _RPT - k)
        pltpu.sync_copy(rows_v.at[pl.ds(0, sz)], acc_sh.at[pl.ds(base + k, sz)])
    plsc.subcore_barrier()

    def body(i, carry):
        cp = pltpu.async_copy(h_hbm.at[src_v.at[i]], rows_v, sem)
        es = []
        for j in range(BB // L):
            s = src_v[i, pl.ds(j * L, L)]
            d = dst_v[i, pl.ds(j * L, L)]
            a = plsc.load_gather(asrc_v, [s]) + plsc.load_gather(adst_v, [d])
            a = jnp.where(a >= 0.0, a, 0.2 * a)
            e = jnp.exp(a - plsc.load_gather(c_v, [d]))
            plsc.addupdate_scatter(dloc_v, [d], e)
            es.append(e)
        cp.wait()
        for j in range(BB // L):
            for q in range(L):
                w = _splat(es[j], q)
                r = j * L + q
                for f in range(DH // L):
                    rows_v[r, pl.ds(f * L, L)] = rows_v[r, pl.ds(f * L, L)] * w
        pltpu.sync_copy(rows_v, acc_sh.at[dst_v.at[i]], add=True)
        return carry

    lax.fori_loop(0, NB, body, 0)
    plsc.subcore_barrier()

    for k in range(0, _RPT, _WCH):
        sz = min(_WCH, _RPT - k)
        pltpu.sync_copy(acc_sh.at[pl.ds(base + k, sz)], rows_v.at[pl.ds(0, sz)])
        pltpu.sync_copy(rows_v.at[pl.ds(0, sz)],
                        agg_hbm.at[cid, pl.ds(base + k, sz)])
    pltpu.sync_copy(dloc_v, dsum_hbm.at[wid])


def _epi2_body(b_ref, a00_ref, a01_ref, d0_ref, a10_ref, a11_ref, d1_ref, o_ref):
    heads = ((a00_ref, a01_ref, d0_ref), (a10_ref, a11_ref, d1_ref))
    for hd, (ah0, ah1, d_ref) in enumerate(heads):
        agg = jnp.concatenate([ah0[...], ah1[...]], axis=-1)
        den = jnp.sum(d_ref[...], axis=0)
        val = (agg[0] + agg[1]) / (den[:, None] + 1e-16)
        o_ref[:, hd * D:(hd + 1) * D] = jnp.maximum(
            val + b_ref[hd * D:(hd + 1) * D][None, :], 0.0)


def _epi1_body(b_ref, a00_ref, a01_ref, d0_ref, o_ref):
    agg = jnp.concatenate([a00_ref[...], a01_ref[...]], axis=-1)
    den = jnp.sum(d0_ref[...], axis=0)
    val = (agg[0] + agg[1]) / (den[:, None] + 1e-16)
    o_ref[...] = jnp.maximum(val + b_ref[...][None, :], 0.0)


def _epilogue(aggs, dsums, b):
    heads = len(dsums)
    R = 2560
    grid = (NP // R,)
    ins = [b]
    in_specs = [pl.BlockSpec((b.shape[0],), lambda i: (0,))]
    for hd in range(heads):
        ins += [aggs[hd][0], aggs[hd][1], dsums[hd]]
        in_specs += [
            pl.BlockSpec((NC, R, DH), lambda i: (0, i, 0)),
            pl.BlockSpec((NC, R, DH), lambda i: (0, i, 0)),
            pl.BlockSpec((NW, R), lambda i: (0, i)),
        ]
    body = _epi2_body if heads == 2 else _epi1_body
    out = pl.pallas_call(
        body,
        grid=grid,
        in_specs=in_specs,
        out_specs=pl.BlockSpec((R, heads * D), lambda i: (i, 0)),
        out_shape=jax.ShapeDtypeStruct((NP, heads * D), jnp.float32),
    )(*ins)
    return out[:N]


def _pool_body(oh_ref, x_ref, o_ref, cnt_ref):
    i = pl.program_id(0)

    @pl.when(i == 0)
    def _():
        o_ref[...] = jnp.zeros_like(o_ref)
        cnt_ref[...] = jnp.zeros_like(cnt_ref)

    oh = oh_ref[...]
    o_ref[...] += jnp.dot(oh, x_ref[...], preferred_element_type=jnp.float32)
    cnt_ref[...] += jnp.broadcast_to(jnp.sum(oh, axis=1, keepdims=True),
                                     cnt_ref.shape)

    @pl.when(i == pl.num_programs(0) - 1)
    def _():
        o_ref[...] = o_ref[...] / jnp.maximum(cnt_ref[...], 1.0)


def _mean_pool(onehot, x):
    bn = 1024
    npad = onehot.shape[1]
    return pl.pallas_call(
        _pool_body,
        grid=(npad // bn,),
        in_specs=[
            pl.BlockSpec((G, bn), lambda i: (0, i)),
            pl.BlockSpec((bn, D), lambda i: (i, 0)),
        ],
        out_specs=pl.BlockSpec((G, D), lambda i: (0, 0)),
        out_shape=jax.ShapeDtypeStruct((G, D), jnp.float32),
        scratch_shapes=[pltpu.VMEM((G, D), jnp.float32)],
    )(onehot, x)


def _gat_layer(x, src_w, dst_w, W, a_src, a_dst, b, heads):
    dout = W.shape[1]
    eye = jnp.eye(heads, dtype=jnp.float32)
    As = (a_src[0][:, :, None] * eye[:, None, :]).reshape(dout, heads)
    Ad = (a_dst[0][:, :, None] * eye[:, None, :]).reshape(dout, heads)
    A = jnp.concatenate([As, Ad], axis=1)
    h2, al = _dense(x, W, A)
    asrc_n = al[:, :heads]
    adst_n = al[:, heads:]
    gms = jnp.max(asrc_n, axis=0)
    c_n = gms[None, :] + adst_n
    c_n = jnp.where(c_n >= 0.0, c_n, 0.2 * c_n)
    pad = NP - N
    aggs, dsums = [], []
    for hd in range(heads):
        asrc_p = jnp.pad(asrc_n[:, hd], (0, pad))
        adst_p = jnp.pad(adst_n[:, hd], (0, pad))
        c_p = jnp.pad(c_n[:, hd], (0, pad))
        halves = []
        dsum = None
        for hf in range(2):
            h_h = h2[:, hd * D + hf * DH:hd * D + (hf + 1) * DH]
            agg, ds = _edge_kernel(src_w, dst_w, asrc_p, adst_p, c_p, h_h)
            halves.append(agg)
            dsum = ds if dsum is None else dsum
        aggs.append(halves)
        dsums.append(dsum)
    return _epilogue(aggs, dsums, b)


def kernel(x, edge_index, batch, W0, asrc0, adst0, b0, W1, asrc1, adst1, b1,
           W2, asrc2, adst2, b2, W3, asrc3, adst3, b3):
    sl = jnp.arange(N, dtype=edge_index.dtype)
    src = jnp.concatenate([edge_index[0], sl])
    dst = jnp.concatenate([edge_index[1], sl])
    src_w = jnp.pad(src, (0, EPAD - E_TOT)).reshape(NW, NB, BB)
    dst_w = jnp.pad(dst, (0, EPAD - E_TOT),
                    constant_values=N).reshape(NW, NB, BB)
    scale = jnp.ones((x.shape[1],), jnp.float32)
    scale = scale.at[0].set(1 / 100.0).at[1].set(1 / 400.0).at[-1].set(1 / 100.0)
    x = x * scale[None, :]
    layers = [(W0, asrc0, adst0, b0, 2), (W1, asrc1, adst1, b1, 2),
              (W2, asrc2, adst2, b2, 2), (W3, asrc3, adst3, b3, 1)]
    for W, a_s, a_d, b, heads in layers:
        x = _gat_layer(x, src_w, dst_w, W, a_s, a_d, b, heads)
    npad = 10240
    batch_p = jnp.pad(batch, (0, npad - N), constant_values=G)
    onehot = (batch_p[None, :] == jnp.arange(G, dtype=batch.dtype)[:, None])
    x_p = jnp.pad(x, ((0, npad - N), (0, 0)))
    return _mean_pool(onehot.astype(jnp.float32), x_p)
